# baseline (speedup 1.0000x reference)
import jax
import jax.numpy as jnp
from jax.experimental import pallas as pl
from jax.experimental.pallas import tpu as pltpu

DAMPING = 0.1
K = 16
NQ = 2
RC = 50


def _pos_add_kernel(x_hbm, pos_vmem, o_hbm, in_buf, out_buf, posf,
                    in_sem, out_sem):
    R, C, E = posf.shape
    N = x_hbm.shape[0] // RC
    PER = R // RC

    posf[...] = jnp.broadcast_to(
        (pos_vmem[...] * DAMPING)[:, None, :], posf.shape)

    def in_copy(i, slot):
        return pltpu.make_async_copy(
            x_hbm.at[pl.ds(i * RC, RC)], in_buf.at[slot], in_sem.at[slot])

    def out_copy(i, slot):
        return pltpu.make_async_copy(
            out_buf.at[slot], o_hbm.at[pl.ds(i * RC, RC)], out_sem.at[slot])

    def start_in(i, slot):
        in_copy(i, slot).start(priority=slot % NQ)

    def start_out(i, slot):
        out_copy(i, slot).start(priority=slot % NQ)

    def compute(i, slot):
        j = jax.lax.rem(i, PER) * RC
        out_buf[slot] = in_buf[slot] + posf[pl.ds(j, RC)]

    for k in range(K):
        start_in(k, k)

    for k in range(K):
        in_copy(k, k).wait()
        compute(k, k)
        start_out(k, k)
        start_in(k + K, k)

    def mid_body(s, carry):
        base = s * K
        for k in range(K):
            i = base + k
            in_copy(i, k).wait()
            out_copy(i - K, k).wait()
            compute(i, k)
            start_out(i, k)
            start_in(i + K, k)
        return carry

    jax.lax.fori_loop(1, N // K - 1, mid_body, 0)

    for k in range(K):
        i = N - K + k
        in_copy(i, k).wait()
        out_copy(i - K, k).wait()
        compute(i, k)
        start_out(i, k)

    for k in range(K):
        out_copy(N - K + k, k).wait()


def kernel(x, pos_table):
    B, R, C, E = x.shape
    x3 = x.reshape(B * R, C, E)
    out = pl.pallas_call(
        _pos_add_kernel,
        in_specs=[
            pl.BlockSpec(memory_space=pl.ANY),
            pl.BlockSpec(memory_space=pltpu.VMEM),
        ],
        out_specs=pl.BlockSpec(memory_space=pl.ANY),
        out_shape=jax.ShapeDtypeStruct((B * R, C, E), x.dtype),
        scratch_shapes=[
            pltpu.VMEM((K, RC, C, E), jnp.float32),
            pltpu.VMEM((K, RC, C, E), jnp.float32),
            pltpu.VMEM((R, C, E), jnp.float32),
            pltpu.SemaphoreType.DMA((K,)),
            pltpu.SemaphoreType.DMA((K,)),
        ],
    )(x3, pos_table)
    return out.reshape(B, R, C, E)


# speedup vs baseline: 1.2479x; 1.2479x over previous
import jax
import jax.numpy as jnp
from jax.experimental import pallas as pl
from jax.experimental.pallas import tpu as pltpu

DAMPING = 0.1
K = 16
NQ = 2
RC = 50


def _pos_add_kernel(x_hbm, pos_vmem, o_hbm, in_buf, out_buf, posf,
                    in_sem, out_sem):
    R, C, E = posf.shape
    B = x_hbm.shape[0]
    PER = R // RC
    N = B * PER

    posf[...] = jnp.broadcast_to(
        (pos_vmem[...] * DAMPING)[:, None, :], posf.shape)

    def in_copy(i, slot):
        b, j = jax.lax.div(i, PER), jax.lax.rem(i, PER)
        return pltpu.make_async_copy(
            x_hbm.at[b, pl.ds(j * RC, RC)], in_buf.at[slot], in_sem.at[slot])

    def out_copy(i, slot):
        b, j = jax.lax.div(i, PER), jax.lax.rem(i, PER)
        return pltpu.make_async_copy(
            out_buf.at[slot], o_hbm.at[b, pl.ds(j * RC, RC)], out_sem.at[slot])

    def start_in(i, slot):
        in_copy(i, slot).start(priority=slot % NQ)

    def start_out(i, slot):
        out_copy(i, slot).start(priority=slot % NQ)

    def compute(i, slot):
        j = jax.lax.rem(i, PER) * RC
        out_buf[slot] = in_buf[slot] + posf[pl.ds(j, RC)]

    for k in range(K):
        start_in(k, k)

    for k in range(K):
        in_copy(k, k).wait()
        compute(k, k)
        start_out(k, k)
        start_in(k + K, k)

    def mid_body(s, carry):
        base = s * K
        for k in range(K):
            i = base + k
            in_copy(i, k).wait()
            out_copy(i - K, k).wait()
            compute(i, k)
            start_out(i, k)
            start_in(i + K, k)
        return carry

    jax.lax.fori_loop(1, N // K - 1, mid_body, 0)

    for k in range(K):
        i = N - K + k
        in_copy(i, k).wait()
        out_copy(i - K, k).wait()
        compute(i, k)
        start_out(i, k)

    for k in range(K):
        out_copy(N - K + k, k).wait()


def kernel(x, pos_table):
    B, R, C, E = x.shape
    return pl.pallas_call(
        _pos_add_kernel,
        in_specs=[
            pl.BlockSpec(memory_space=pl.ANY),
            pl.BlockSpec(memory_space=pltpu.VMEM),
        ],
        out_specs=pl.BlockSpec(memory_space=pl.ANY),
        out_shape=jax.ShapeDtypeStruct(x.shape, x.dtype),
        scratch_shapes=[
            pltpu.VMEM((K, RC, C, E), jnp.float32),
            pltpu.VMEM((K, RC, C, E), jnp.float32),
            pltpu.VMEM((R, C, E), jnp.float32),
            pltpu.SemaphoreType.DMA((K,)),
            pltpu.SemaphoreType.DMA((K,)),
        ],
    )(x, pos_table)
